# baseline (speedup 1.0000x reference)
import functools

import jax
import jax.numpy as jnp
from jax import lax
from jax.experimental import pallas as pl
from jax.experimental.pallas import tpu as pltpu
from jax.experimental.pallas import tpu_sc as plsc

N = 10000
M = 16
F = 128
NF = 16
G2 = 2 * F
MGL = 512
B = 8
APC = 1250
U = 1024
A_T = 400
E_T = A_T * M
GRID = N // A_T
NE = N * M
NW = 32
CH = 40
NCH = NE // (NW * CH)


def _sc_gather(table, idxp, nfull, tail):
    nw, nrows, chp = idxp.shape
    ch = 128
    rows_per_w = nfull * ch + tail
    total = nfull + (1 if tail else 0)
    mesh = plsc.VectorSubcoreMesh(core_axis_name="c", subcore_axis_name="s")

    @functools.partial(
        pl.kernel,
        out_type=jax.ShapeDtypeStruct((nw * rows_per_w, F), jnp.float32),
        mesh=mesh,
        scratch_types=[
            pltpu.VMEM((nrows, chp), jnp.int32),
            pltpu.VMEM((4, ch, F), jnp.float32),
            pltpu.SemaphoreType.DMA((4,)),
            pltpu.SemaphoreType.DMA((4,)),
        ],
    )
    def k(table_hbm, idx_hbm, out_hbm, idx_v, buf, sem_g, sem_o):
        wid = lax.axis_index("s") * 2 + lax.axis_index("c")
        base = wid * rows_per_w
        pltpu.sync_copy(idx_hbm.at[wid], idx_v)
        pltpu.async_copy(table_hbm.at[idx_v.at[0]], buf.at[0], sem_g.at[0])
        if total > 1:
            pltpu.async_copy(table_hbm.at[idx_v.at[1]], buf.at[1],
                             sem_g.at[1])

        def step(j, carry):
            slot = lax.rem(j, 4)
            pltpu.make_async_copy(
                table_hbm.at[idx_v.at[j]], buf.at[slot],
                sem_g.at[slot]).wait()
            pltpu.async_copy(buf.at[slot],
                             out_hbm.at[pl.ds(base + j * ch, ch)],
                             sem_o.at[slot])

            @pl.when(j >= 2)
            def _():
                oslot = lax.rem(j - 2, 4)
                pltpu.make_async_copy(
                    buf.at[oslot],
                    out_hbm.at[pl.ds(base + (j - 2) * ch, ch)],
                    sem_o.at[oslot]).wait()

            @pl.when(j + 2 < total)
            def _():
                nslot = lax.rem(j + 2, 4)
                pltpu.async_copy(table_hbm.at[idx_v.at[j + 2]],
                                 buf.at[nslot], sem_g.at[nslot])

            return carry

        lax.fori_loop(0, nfull, step, 0)
        if tail:
            tslot = nfull % 4
            pltpu.make_async_copy(table_hbm.at[idx_v.at[nfull]],
                                  buf.at[tslot], sem_g.at[tslot]).wait()
            pltpu.async_copy(buf.at[tslot, pl.ds(0, tail)],
                             out_hbm.at[pl.ds(base + nfull * ch, tail)],
                             sem_o.at[tslot])
            pltpu.make_async_copy(
                buf.at[tslot, pl.ds(0, tail)],
                out_hbm.at[pl.ds(base + nfull * ch, tail)],
                sem_o.at[tslot]).wait()
        for jt in range(max(nfull - 2, 0), nfull):
            pltpu.make_async_copy(
                buf.at[jt % 4],
                out_hbm.at[pl.ds(base + jt * ch, ch)],
                sem_o.at[jt % 4]).wait()

    return k(table, idxp)


def _ka0_body(ids_ref, emb_ref, w_ref, b_ref, af_ref, s_ref):
    ids = ids_ref[...]
    t = lax.broadcasted_iota(jnp.int32, (A_T, F), 1)
    oh = (ids == t).astype(jnp.float32)
    af = jnp.dot(oh, emb_ref[...], preferred_element_type=jnp.float32,
                 precision=lax.Precision.HIGHEST)
    af_ref[...] = af
    s_ref[...] = jnp.dot(af, w_ref[...],
                         preferred_element_type=jnp.float32,
                 precision=lax.Precision.DEFAULT) + b_ref[...]


def _ka_body(af_ref, ns_ref, sums2_ref, g2_ref, b2_ref, w_ref, b_ref,
             afo_ref, s_ref):
    mu = sums2_ref[0:1, :] / N
    var = sums2_ref[1:2, :] / N - mu * mu
    a = g2_ref[...] * lax.rsqrt(var + 1e-5)
    c = b2_ref[...] - mu * a
    af = jax.nn.softplus(af_ref[...] + ns_ref[...] * a + c)
    afo_ref[...] = af
    s_ref[...] = jnp.dot(af, w_ref[...],
                         preferred_element_type=jnp.float32,
                 precision=lax.Precision.DEFAULT) + b_ref[...]


def _kf_body(af_ref, ns_ref, sums2_ref, g2_ref, b2_ref, w_ref, b_ref,
             out_ref):
    mu = sums2_ref[0:1, :] / N
    var = sums2_ref[1:2, :] / N - mu * mu
    a = g2_ref[...] * lax.rsqrt(var + 1e-5)
    c = b2_ref[...] - mu * a
    af = jax.nn.softplus(af_ref[...] + ns_ref[...] * a + c)
    out_ref[...] = jnp.dot(af, w_ref[...],
                           preferred_element_type=jnp.float32,
                 precision=lax.Precision.DEFAULT) + b_ref[...]


def _xe(g_ref, nbrf_ref, wn, we):
    x = jnp.dot(g_ref[...], wn, preferred_element_type=jnp.float32)
    e = jnp.dot(nbrf_ref[...].reshape(E_T, NF), we,
                preferred_element_type=jnp.float32)
    return x + e


def _kc_body(g_ref, s_ref, nbrf_ref, wn_ref, we_ref, sums_ref, acc_ref):
    step = pl.program_id(0)

    @pl.when(step == 0)
    def _():
        acc_ref[...] = jnp.zeros_like(acc_ref)

    xe = _xe(g_ref, nbrf_ref, wn_ref[...], we_ref[...])
    sblk = s_ref[...]
    t = jnp.sum(xe.reshape(A_T, M, G2), axis=1)
    s1 = (jnp.sum(xe, axis=0, keepdims=True)
          + M * jnp.sum(sblk, axis=0, keepdims=True))
    s2 = (jnp.sum(xe * xe, axis=0, keepdims=True)
          + 2.0 * jnp.sum(sblk * t, axis=0, keepdims=True)
          + M * jnp.sum(sblk * sblk, axis=0, keepdims=True))
    acc_ref[...] += jnp.concatenate([s1, s2], axis=0)

    @pl.when(step == GRID - 1)
    def _():
        sums_ref[...] = acc_ref[...]


def _kd_body(g_ref, s_ref, nbrf_ref, wn_ref, we_ref, sums1_ref, g1_ref,
             b1_ref, ns_ref, sums2_ref, acc_ref):
    step = pl.program_id(0)

    @pl.when(step == 0)
    def _():
        acc_ref[...] = jnp.zeros_like(acc_ref)

    mu = sums1_ref[0:1, :] / NE
    var = sums1_ref[1:2, :] / NE - mu * mu
    a = g1_ref[...] * lax.rsqrt(var + 1e-5)
    c = b1_ref[...] - mu * a
    xe = _xe(g_ref, nbrf_ref, wn_ref[...] * a, we_ref[...] * a)
    sp = s_ref[...] * a + c
    norm = (xe.reshape(A_T, M, G2)
            + sp.reshape(A_T, 1, G2)).reshape(E_T, G2)
    contrib = jax.nn.sigmoid(norm[:, :F]) * jax.nn.softplus(norm[:, F:])
    ns = jnp.sum(contrib.reshape(A_T, M, F), axis=1)
    ns_ref[...] = ns
    s1 = jnp.sum(ns, axis=0, keepdims=True)
    s2 = jnp.sum(ns * ns, axis=0, keepdims=True)
    acc_ref[...] += jnp.concatenate([s1, s2], axis=0)

    @pl.when(step == GRID - 1)
    def _():
        sums2_ref[...] = acc_ref[...]


def _kr_body(cnt_ref, cntT_ref, uniT_ref, cai_ref, out_ref):
    cnt = cnt_ref[...].reshape(1, U)
    cntT = cntT_ref[...].reshape(U, 1)
    uniT = uniT_ref[...].reshape(U, 1)
    cai = cai_ref[...].reshape(1, 1280)
    jl = lax.broadcasted_iota(jnp.int32, (1, U), 1)
    key_lane = cnt * U + (U - 1) - jl
    il = lax.broadcasted_iota(jnp.int32, (U, 1), 0)
    key_sub = cntT * U + (U - 1) - il
    rank = jnp.sum((key_lane > key_sub).astype(jnp.int32),
                   axis=1, keepdims=True)
    tt = lax.broadcasted_iota(jnp.int32, (U, 1280), 1)
    match = jnp.where(tt == uniT, cai, 0.0)
    comp = jnp.sum(match, axis=1, keepdims=True)
    pp = lax.broadcasted_iota(jnp.int32, (U, MGL), 1)
    frow = jnp.sum(jnp.where(rank == pp, comp, 0.0),
                   axis=0, keepdims=True)
    out_ref[...] = frow.astype(jnp.int32).reshape(1, 1, MGL)


def _kmask_body(rows_ref, mask_ref):
    s = jnp.sum(rows_ref[...], axis=1, keepdims=True)
    mask_ref[...] = (s != 0.0).astype(jnp.float32)


_SEQ = pltpu.CompilerParams(dimension_semantics=("arbitrary",))


def _full(shape):
    return pl.BlockSpec(shape, lambda i: tuple(0 for _ in shape))


def _ka0(ids2, emb_pad, w_self, bias):
    return pl.pallas_call(
        _ka0_body,
        grid=(GRID,),
        in_specs=[pl.BlockSpec((A_T, 1), lambda i: (i, 0)),
                  _full((F, F)), _full((F, G2)), _full((1, G2))],
        out_specs=[pl.BlockSpec((A_T, F), lambda i: (i, 0)),
                   pl.BlockSpec((A_T, G2), lambda i: (i, 0))],
        out_shape=[jax.ShapeDtypeStruct((N, F), jnp.float32),
                   jax.ShapeDtypeStruct((N, G2), jnp.float32)],
        compiler_params=_SEQ,
    )(ids2, emb_pad, w_self, bias)


def _ka(af, ns, sums2, g2, b2, w_self, bias):
    return pl.pallas_call(
        _ka_body,
        grid=(GRID,),
        in_specs=[pl.BlockSpec((A_T, F), lambda i: (i, 0)),
                  pl.BlockSpec((A_T, F), lambda i: (i, 0)),
                  _full((2, F)), _full((1, F)), _full((1, F)),
                  _full((F, G2)), _full((1, G2))],
        out_specs=[pl.BlockSpec((A_T, F), lambda i: (i, 0)),
                   pl.BlockSpec((A_T, G2), lambda i: (i, 0))],
        out_shape=[jax.ShapeDtypeStruct((N, F), jnp.float32),
                   jax.ShapeDtypeStruct((N, G2), jnp.float32)],
        compiler_params=_SEQ,
    )(af, ns, sums2, g2, b2, w_self, bias)


def _kf(af, ns, sums2, g2, b2, out_w, out_b):
    return pl.pallas_call(
        _kf_body,
        grid=(GRID,),
        in_specs=[pl.BlockSpec((A_T, F), lambda i: (i, 0)),
                  pl.BlockSpec((A_T, F), lambda i: (i, 0)),
                  _full((2, F)), _full((1, F)), _full((1, F)),
                  _full((F, F)), _full((1, F))],
        out_specs=pl.BlockSpec((A_T, F), lambda i: (i, 0)),
        out_shape=jax.ShapeDtypeStruct((N, F), jnp.float32),
        compiler_params=_SEQ,
    )(af, ns, sums2, g2, b2, out_w, out_b)


def _kc(g, s, nbrf, w_nbr, w_e):
    return pl.pallas_call(
        _kc_body,
        grid=(GRID,),
        in_specs=[pl.BlockSpec((E_T, F), lambda i: (i, 0)),
                  pl.BlockSpec((A_T, G2), lambda i: (i, 0)),
                  pl.BlockSpec((A_T, M, NF), lambda i: (i, 0, 0)),
                  _full((F, G2)), _full((NF, G2))],
        out_specs=_full((2, G2)),
        out_shape=jax.ShapeDtypeStruct((2, G2), jnp.float32),
        scratch_shapes=[pltpu.VMEM((2, G2), jnp.float32)],
        compiler_params=_SEQ,
    )(g, s, nbrf, w_nbr, w_e)


def _kd(g, s, nbrf, w_nbr, w_e, sums1, g1, b1):
    return pl.pallas_call(
        _kd_body,
        grid=(GRID,),
        in_specs=[pl.BlockSpec((E_T, F), lambda i: (i, 0)),
                  pl.BlockSpec((A_T, G2), lambda i: (i, 0)),
                  pl.BlockSpec((A_T, M, NF), lambda i: (i, 0, 0)),
                  _full((F, G2)), _full((NF, G2)),
                  _full((2, G2)), _full((1, G2)), _full((1, G2))],
        out_specs=[pl.BlockSpec((A_T, F), lambda i: (i, 0)),
                   _full((2, F))],
        out_shape=[jax.ShapeDtypeStruct((N, F), jnp.float32),
                   jax.ShapeDtypeStruct((2, F), jnp.float32)],
        scratch_shapes=[pltpu.VMEM((2, F), jnp.float32)],
        compiler_params=_SEQ,
    )(g, s, nbrf, w_nbr, w_e, sums1, g1, b1)


def _kr(cnt3, cntT3, uni3, caiT3):
    return pl.pallas_call(
        _kr_body,
        grid=(B,),
        in_specs=[pl.BlockSpec((1, 1, U), lambda b: (b, 0, 0)),
                  pl.BlockSpec((1, U, 1), lambda b: (b, 0, 0)),
                  pl.BlockSpec((1, U, 1), lambda b: (b, 0, 0)),
                  pl.BlockSpec((1, 1, 1280), lambda b: (b, 0, 0))],
        out_specs=pl.BlockSpec((1, 1, MGL), lambda b: (b, 0, 0)),
        out_shape=jax.ShapeDtypeStruct((B, 1, MGL), jnp.int32),
        compiler_params=_SEQ,
    )(cnt3, cntT3, uni3, caiT3)


def _kmask(rows):
    return pl.pallas_call(
        _kmask_body,
        out_shape=jax.ShapeDtypeStruct((B * MGL, 1), jnp.float32),
    )(rows)


def kernel(atom_num, nbr_idx, nbr_fea, crystal_atom_idx, uni_idx, uni_count,
           emb, fc_w, fc_b, bn1_g, bn1_b, bn2_g, bn2_b, out_w, out_b):
    ids2 = atom_num.astype(jnp.int32).reshape(N, 1)
    emb_pad = jnp.pad(emb, ((0, F - emb.shape[0]), (0, 0)))
    idx3 = jnp.pad(nbr_idx.astype(jnp.int32).reshape(NW, NE // NW),
                   ((0, 0), (0, 40 * 128 - NE // NW))).reshape(NW, 40, 128)
    nbrf = nbr_fea

    af = None
    ns = None
    sums2 = None
    for i in range(3):
        w_self = fc_w[i, :F, :]
        w_nbr = fc_w[i, F:2 * F, :]
        w_e = fc_w[i, 2 * F:, :]
        bias = fc_b[i].reshape(1, G2)
        g1 = bn1_g[i].reshape(1, G2)
        b1 = bn1_b[i].reshape(1, G2)
        if i == 0:
            af, s = _ka0(ids2, emb_pad, w_self, bias)
        else:
            af, s = _ka(af, ns, sums2, bn2_g[i - 1].reshape(1, F),
                        bn2_b[i - 1].reshape(1, F), w_self, bias)
        g = _sc_gather(af, idx3, 39, 8)
        sums1 = _kc(g, s, nbrf, w_nbr, w_e)
        ns, sums2 = _kd(g, s, nbrf, w_nbr, w_e, sums1, g1, b1)

    out_af = _kf(af, ns, sums2, bn2_g[2].reshape(1, F),
                 bn2_b[2].reshape(1, F), out_w, out_b.reshape(1, F))

    cnt = uni_count.astype(jnp.int32)
    cai3 = jnp.pad(crystal_atom_idx.astype(jnp.float32),
                   ((0, 0), (0, 1280 - APC))).reshape(B, 1, 1280)
    frow = _kr(cnt.reshape(B, 1, U), cnt.reshape(B, U, 1),
               uni_idx.astype(jnp.int32).reshape(B, U, 1), cai3)
    frow_pad = jnp.pad(frow.reshape(NW, 1, B * MGL // NW),
                       ((0, 0), (0, 7), (0, 0)))
    rows = _sc_gather(out_af, frow_pad, 1, 0)
    mask = _kmask(rows)
    return rows.reshape(B, MGL, F), mask.reshape(B, MGL)


# speedup vs baseline: 1.4370x; 1.4370x over previous
import functools

import jax
import jax.numpy as jnp
from jax import lax
from jax.experimental import pallas as pl
from jax.experimental.pallas import tpu as pltpu
from jax.experimental.pallas import tpu_sc as plsc

N = 10000
M = 16
F = 128
NF = 16
G2 = 2 * F
MGL = 512
B = 8
APC = 1250
U = 1024
A_T = 400
E_T = A_T * M
GRID = N // A_T
NE = N * M
NW = 32
CH = 40
NCH = NE // (NW * CH)


def _sc_gather(table, idx3, nch, ch):
    nw, nrows, chp = idx3.shape
    rows_per_w = nch * ch
    mesh = plsc.VectorSubcoreMesh(core_axis_name="c", subcore_axis_name="s")

    @functools.partial(
        pl.kernel,
        out_type=jax.ShapeDtypeStruct((nw * rows_per_w, F), jnp.float32),
        mesh=mesh,
        scratch_types=[
            pltpu.VMEM((nrows, chp), jnp.int32),
            pltpu.VMEM((4, ch, F), jnp.float32),
            pltpu.SemaphoreType.DMA((4,)),
            pltpu.SemaphoreType.DMA((4,)),
        ],
    )
    def k(table_hbm, idx_hbm, out_hbm, idx_v, buf, sem_g, sem_o):
        wid = lax.axis_index("s") * 2 + lax.axis_index("c")
        base = wid * rows_per_w
        pltpu.sync_copy(idx_hbm.at[wid], idx_v)
        pltpu.async_copy(table_hbm.at[idx_v.at[0, pl.ds(0, ch)]], buf.at[0],
                         sem_g.at[0])
        if nch > 1:
            pltpu.async_copy(table_hbm.at[idx_v.at[1, pl.ds(0, ch)]],
                             buf.at[1], sem_g.at[1])

        def step(j, carry):
            slot = lax.rem(j, 4)
            pltpu.make_async_copy(
                table_hbm.at[idx_v.at[j, pl.ds(0, ch)]], buf.at[slot],
                sem_g.at[slot]).wait()
            pltpu.async_copy(buf.at[slot],
                             out_hbm.at[pl.ds(base + j * ch, ch)],
                             sem_o.at[slot])

            @pl.when(j >= 2)
            def _():
                oslot = lax.rem(j - 2, 4)
                pltpu.make_async_copy(
                    buf.at[oslot],
                    out_hbm.at[pl.ds(base + (j - 2) * ch, ch)],
                    sem_o.at[oslot]).wait()

            @pl.when(j + 2 < nch)
            def _():
                nslot = lax.rem(j + 2, 4)
                pltpu.async_copy(
                    table_hbm.at[idx_v.at[j + 2, pl.ds(0, ch)]],
                    buf.at[nslot], sem_g.at[nslot])

            return carry

        lax.fori_loop(0, nch, step, 0)
        for jt in range(max(nch - 2, 0), nch):
            pltpu.make_async_copy(
                buf.at[jt % 4],
                out_hbm.at[pl.ds(base + jt * ch, ch)],
                sem_o.at[jt % 4]).wait()

    return k(table, idx3)


def _ka0_body(ids_ref, emb_ref, w_ref, b_ref, af_ref, s_ref):
    ids = ids_ref[...]
    t = lax.broadcasted_iota(jnp.int32, (A_T, F), 1)
    oh = (ids == t).astype(jnp.float32)
    af = jnp.dot(oh, emb_ref[...], preferred_element_type=jnp.float32,
                 precision=lax.Precision.HIGHEST)
    af_ref[...] = af
    s_ref[...] = jnp.dot(af, w_ref[...],
                         preferred_element_type=jnp.float32,
                 precision=lax.Precision.DEFAULT) + b_ref[...]


def _ka_body(af_ref, ns_ref, sums2_ref, g2_ref, b2_ref, w_ref, b_ref,
             afo_ref, s_ref):
    mu = sums2_ref[0:1, :] / N
    var = sums2_ref[1:2, :] / N - mu * mu
    a = g2_ref[...] * lax.rsqrt(var + 1e-5)
    c = b2_ref[...] - mu * a
    af = jax.nn.softplus(af_ref[...] + ns_ref[...] * a + c)
    afo_ref[...] = af
    s_ref[...] = jnp.dot(af, w_ref[...],
                         preferred_element_type=jnp.float32,
                 precision=lax.Precision.DEFAULT) + b_ref[...]


def _kf_body(af_ref, ns_ref, sums2_ref, g2_ref, b2_ref, w_ref, b_ref,
             out_ref):
    mu = sums2_ref[0:1, :] / N
    var = sums2_ref[1:2, :] / N - mu * mu
    a = g2_ref[...] * lax.rsqrt(var + 1e-5)
    c = b2_ref[...] - mu * a
    af = jax.nn.softplus(af_ref[...] + ns_ref[...] * a + c)
    out_ref[...] = jnp.dot(af, w_ref[...],
                           preferred_element_type=jnp.float32,
                 precision=lax.Precision.DEFAULT) + b_ref[...]


def _xe(g_ref, nbrf_ref, wn, we):
    x = jnp.dot(g_ref[...], wn, preferred_element_type=jnp.float32)
    e = jnp.dot(nbrf_ref[...].reshape(E_T, NF), we,
                preferred_element_type=jnp.float32)
    return x + e


def _kc_body(g_ref, s_ref, nbrf_ref, wn_ref, we_ref, sums_ref, acc_ref):
    step = pl.program_id(0)

    @pl.when(step == 0)
    def _():
        acc_ref[...] = jnp.zeros_like(acc_ref)

    xe = _xe(g_ref, nbrf_ref, wn_ref[...], we_ref[...])
    sblk = s_ref[...]
    t = jnp.sum(xe.reshape(A_T, M, G2), axis=1)
    s1 = (jnp.sum(xe, axis=0, keepdims=True)
          + M * jnp.sum(sblk, axis=0, keepdims=True))
    s2 = (jnp.sum(xe * xe, axis=0, keepdims=True)
          + 2.0 * jnp.sum(sblk * t, axis=0, keepdims=True)
          + M * jnp.sum(sblk * sblk, axis=0, keepdims=True))
    acc_ref[...] += jnp.concatenate([s1, s2], axis=0)

    @pl.when(step == GRID - 1)
    def _():
        sums_ref[...] = acc_ref[...]


def _kd_body(g_ref, s_ref, nbrf_ref, wn_ref, we_ref, sums1_ref, g1_ref,
             b1_ref, ns_ref, sums2_ref, acc_ref):
    step = pl.program_id(0)

    @pl.when(step == 0)
    def _():
        acc_ref[...] = jnp.zeros_like(acc_ref)

    mu = sums1_ref[0:1, :] / NE
    var = sums1_ref[1:2, :] / NE - mu * mu
    a = g1_ref[...] * lax.rsqrt(var + 1e-5)
    c = b1_ref[...] - mu * a
    xe = _xe(g_ref, nbrf_ref, wn_ref[...] * a, we_ref[...] * a)
    sp = s_ref[...] * a + c
    norm = (xe.reshape(A_T, M, G2)
            + sp.reshape(A_T, 1, G2)).reshape(E_T, G2)
    contrib = jax.nn.sigmoid(norm[:, :F]) * jax.nn.softplus(norm[:, F:])
    ns = jnp.sum(contrib.reshape(A_T, M, F), axis=1)
    ns_ref[...] = ns
    s1 = jnp.sum(ns, axis=0, keepdims=True)
    s2 = jnp.sum(ns * ns, axis=0, keepdims=True)
    acc_ref[...] += jnp.concatenate([s1, s2], axis=0)

    @pl.when(step == GRID - 1)
    def _():
        sums2_ref[...] = acc_ref[...]


def _kr_body(cnt_ref, cntT_ref, uniT_ref, cai_ref, out_ref):
    cnt = cnt_ref[...].reshape(1, U)
    cntT = cntT_ref[...].reshape(U, 1)
    uniT = uniT_ref[...].reshape(U, 1)
    cai = cai_ref[...].reshape(1, 1280)
    jl = lax.broadcasted_iota(jnp.int32, (1, U), 1)
    key_lane = cnt * U + (U - 1) - jl
    il = lax.broadcasted_iota(jnp.int32, (U, 1), 0)
    key_sub = cntT * U + (U - 1) - il
    rank = jnp.sum((key_lane > key_sub).astype(jnp.int32),
                   axis=1, keepdims=True)
    tt = lax.broadcasted_iota(jnp.int32, (U, 1280), 1)
    match = jnp.where(tt == uniT, cai, 0.0)
    comp = jnp.sum(match, axis=1, keepdims=True)
    pp = lax.broadcasted_iota(jnp.int32, (U, MGL), 1)
    frow = jnp.sum(jnp.where(rank == pp, comp, 0.0),
                   axis=0, keepdims=True)
    out_ref[...] = frow.astype(jnp.int32).reshape(1, 1, MGL)


def _kmask_body(rows_ref, mask_ref):
    s = jnp.sum(rows_ref[...], axis=1, keepdims=True)
    mask_ref[...] = (s != 0.0).astype(jnp.float32)


_SEQ = pltpu.CompilerParams(dimension_semantics=("arbitrary",))


def _full(shape):
    return pl.BlockSpec(shape, lambda i: tuple(0 for _ in shape))


def _ka0(ids2, emb_pad, w_self, bias):
    return pl.pallas_call(
        _ka0_body,
        grid=(GRID,),
        in_specs=[pl.BlockSpec((A_T, 1), lambda i: (i, 0)),
                  _full((F, F)), _full((F, G2)), _full((1, G2))],
        out_specs=[pl.BlockSpec((A_T, F), lambda i: (i, 0)),
                   pl.BlockSpec((A_T, G2), lambda i: (i, 0))],
        out_shape=[jax.ShapeDtypeStruct((N, F), jnp.float32),
                   jax.ShapeDtypeStruct((N, G2), jnp.float32)],
        compiler_params=_SEQ,
    )(ids2, emb_pad, w_self, bias)


def _ka(af, ns, sums2, g2, b2, w_self, bias):
    return pl.pallas_call(
        _ka_body,
        grid=(GRID,),
        in_specs=[pl.BlockSpec((A_T, F), lambda i: (i, 0)),
                  pl.BlockSpec((A_T, F), lambda i: (i, 0)),
                  _full((2, F)), _full((1, F)), _full((1, F)),
                  _full((F, G2)), _full((1, G2))],
        out_specs=[pl.BlockSpec((A_T, F), lambda i: (i, 0)),
                   pl.BlockSpec((A_T, G2), lambda i: (i, 0))],
        out_shape=[jax.ShapeDtypeStruct((N, F), jnp.float32),
                   jax.ShapeDtypeStruct((N, G2), jnp.float32)],
        compiler_params=_SEQ,
    )(af, ns, sums2, g2, b2, w_self, bias)


def _kf(af, ns, sums2, g2, b2, out_w, out_b):
    return pl.pallas_call(
        _kf_body,
        grid=(GRID,),
        in_specs=[pl.BlockSpec((A_T, F), lambda i: (i, 0)),
                  pl.BlockSpec((A_T, F), lambda i: (i, 0)),
                  _full((2, F)), _full((1, F)), _full((1, F)),
                  _full((F, F)), _full((1, F))],
        out_specs=pl.BlockSpec((A_T, F), lambda i: (i, 0)),
        out_shape=jax.ShapeDtypeStruct((N, F), jnp.float32),
        compiler_params=_SEQ,
    )(af, ns, sums2, g2, b2, out_w, out_b)


def _kc(g, s, nbrf, w_nbr, w_e):
    return pl.pallas_call(
        _kc_body,
        grid=(GRID,),
        in_specs=[pl.BlockSpec((E_T, F), lambda i: (i, 0)),
                  pl.BlockSpec((A_T, G2), lambda i: (i, 0)),
                  pl.BlockSpec((A_T, M, NF), lambda i: (i, 0, 0)),
                  _full((F, G2)), _full((NF, G2))],
        out_specs=_full((2, G2)),
        out_shape=jax.ShapeDtypeStruct((2, G2), jnp.float32),
        scratch_shapes=[pltpu.VMEM((2, G2), jnp.float32)],
        compiler_params=_SEQ,
    )(g, s, nbrf, w_nbr, w_e)


def _kd(g, s, nbrf, w_nbr, w_e, sums1, g1, b1):
    return pl.pallas_call(
        _kd_body,
        grid=(GRID,),
        in_specs=[pl.BlockSpec((E_T, F), lambda i: (i, 0)),
                  pl.BlockSpec((A_T, G2), lambda i: (i, 0)),
                  pl.BlockSpec((A_T, M, NF), lambda i: (i, 0, 0)),
                  _full((F, G2)), _full((NF, G2)),
                  _full((2, G2)), _full((1, G2)), _full((1, G2))],
        out_specs=[pl.BlockSpec((A_T, F), lambda i: (i, 0)),
                   _full((2, F))],
        out_shape=[jax.ShapeDtypeStruct((N, F), jnp.float32),
                   jax.ShapeDtypeStruct((2, F), jnp.float32)],
        scratch_shapes=[pltpu.VMEM((2, F), jnp.float32)],
        compiler_params=_SEQ,
    )(g, s, nbrf, w_nbr, w_e, sums1, g1, b1)


def _kr(cnt3, cntT3, uni3, caiT3):
    return pl.pallas_call(
        _kr_body,
        grid=(B,),
        in_specs=[pl.BlockSpec((1, 1, U), lambda b: (b, 0, 0)),
                  pl.BlockSpec((1, U, 1), lambda b: (b, 0, 0)),
                  pl.BlockSpec((1, U, 1), lambda b: (b, 0, 0)),
                  pl.BlockSpec((1, 1, 1280), lambda b: (b, 0, 0))],
        out_specs=pl.BlockSpec((1, 1, MGL), lambda b: (b, 0, 0)),
        out_shape=jax.ShapeDtypeStruct((B, 1, MGL), jnp.int32),
        compiler_params=_SEQ,
    )(cnt3, cntT3, uni3, caiT3)


def _kmask(rows):
    return pl.pallas_call(
        _kmask_body,
        out_shape=jax.ShapeDtypeStruct((B * MGL, 1), jnp.float32),
    )(rows)


def kernel(atom_num, nbr_idx, nbr_fea, crystal_atom_idx, uni_idx, uni_count,
           emb, fc_w, fc_b, bn1_g, bn1_b, bn2_g, bn2_b, out_w, out_b):
    ids2 = atom_num.astype(jnp.int32).reshape(N, 1)
    emb_pad = jnp.pad(emb, ((0, F - emb.shape[0]), (0, 0)))
    idx3 = jnp.pad(nbr_idx.astype(jnp.int32).reshape(NW, NCH, CH),
                   ((0, 0), (0, 128 - NCH), (0, 128 - CH)))
    nbrf = nbr_fea

    af = None
    ns = None
    sums2 = None
    for i in range(3):
        w_self = fc_w[i, :F, :]
        w_nbr = fc_w[i, F:2 * F, :]
        w_e = fc_w[i, 2 * F:, :]
        bias = fc_b[i].reshape(1, G2)
        g1 = bn1_g[i].reshape(1, G2)
        b1 = bn1_b[i].reshape(1, G2)
        if i == 0:
            af, s = _ka0(ids2, emb_pad, w_self, bias)
        else:
            af, s = _ka(af, ns, sums2, bn2_g[i - 1].reshape(1, F),
                        bn2_b[i - 1].reshape(1, F), w_self, bias)
        g = _sc_gather(af, idx3, NCH, CH)
        sums1 = _kc(g, s, nbrf, w_nbr, w_e)
        ns, sums2 = _kd(g, s, nbrf, w_nbr, w_e, sums1, g1, b1)

    out_af = _kf(af, ns, sums2, bn2_g[2].reshape(1, F),
                 bn2_b[2].reshape(1, F), out_w, out_b.reshape(1, F))

    cnt = uni_count.astype(jnp.int32)
    cai3 = jnp.pad(crystal_atom_idx.astype(jnp.float32),
                   ((0, 0), (0, 1280 - APC))).reshape(B, 1, 1280)
    frow = _kr(cnt.reshape(B, 1, U), cnt.reshape(B, U, 1),
               uni_idx.astype(jnp.int32).reshape(B, U, 1), cai3)
    frow_pad = jnp.pad(frow.reshape(NW, 1, B * MGL // NW),
                       ((0, 0), (0, 7), (0, 0)))
    rows = _sc_gather(out_af, frow_pad, 1, B * MGL // NW)
    mask = _kmask(rows)
    return rows.reshape(B, MGL, F), mask.reshape(B, MGL)
